# initial kernel scaffold (unmeasured)
import jax
import jax.numpy as jnp
from jax import lax
from jax.experimental import pallas as pl
from jax.experimental.pallas import tpu as pltpu

N_DEV = 4
F8 = jnp.float8_e4m3fn


def kernel(x, w_mat, scale_x, scale_w):
    m_total, k_loc = x.shape
    k_total, n = w_mat.shape
    m_loc = m_total // N_DEV

    def body(x_ref, w_ref, sx_ref, sw_ref, out_ref,
             x8_ref, recv_ref, send_sems, recv_sems):
        my = lax.axis_index("i")

        for b in range(N_DEV):
            x8_ref[b, :, :] = x_ref[b * m_loc:(b + 1) * m_loc, :].astype(F8)

        barrier = pltpu.get_barrier_semaphore()
        for off in range(1, N_DEV):
            pl.semaphore_signal(
                barrier, inc=1,
                device_id=((my + off) % N_DEV,),
                device_id_type=pl.DeviceIdType.MESH)
        pl.semaphore_wait(barrier, N_DEV - 1)

        sends = []
        for off in range(1, N_DEV):
            tgt = (my + off) % N_DEV
            rdma = pltpu.make_async_remote_copy(
                src_ref=x8_ref.at[tgt],
                dst_ref=recv_ref.at[my],
                send_sem=send_sems.at[off - 1],
                recv_sem=recv_sems.at[my],
                device_id=(tgt,),
                device_id_type=pl.DeviceIdType.MESH)
            rdma.start()
            sends.append(rdma)

        out_ref[:, :] = lax.dot_general(
            x8_ref[my], w_ref[pl.ds(my * k_loc, k_loc), :].astype(F8),
            (((1,), (0,)), ((), ())),
            preferred_element_type=jnp.float32)

        for off in (1, 3, 2):
            src = (my + off) % N_DEV
            recv = pltpu.make_async_remote_copy(
                src_ref=x8_ref.at[0],
                dst_ref=recv_ref.at[src],
                send_sem=send_sems.at[0],
                recv_sem=recv_sems.at[src],
                device_id=(src,),
                device_id_type=pl.DeviceIdType.MESH)
            recv.wait_recv()
            out_ref[:, :] += lax.dot_general(
                recv_ref[src], w_ref[pl.ds(src * k_loc, k_loc), :].astype(F8),
                (((1,), (0,)), ((), ())),
                preferred_element_type=jnp.float32)

        y = out_ref[:, :] * (sx_ref[0] * sw_ref[0])
        out_ref[:, :] = y * (1.0 / (1.0 + jnp.exp(-y)))

        for rdma in sends:
            rdma.wait_send()

    return pl.pallas_call(
        body,
        out_shape=jax.ShapeDtypeStruct((m_loc, n), jnp.float32),
        in_specs=[
            pl.BlockSpec(memory_space=pltpu.VMEM),
            pl.BlockSpec(memory_space=pltpu.VMEM),
            pl.BlockSpec(memory_space=pltpu.SMEM),
            pl.BlockSpec(memory_space=pltpu.SMEM),
        ],
        out_specs=pl.BlockSpec(memory_space=pltpu.VMEM),
        scratch_shapes=[
            pltpu.VMEM((N_DEV, m_loc, k_loc), F8),
            pltpu.VMEM((N_DEV, m_loc, k_loc), F8),
            pltpu.SemaphoreType.DMA((N_DEV - 1,)),
            pltpu.SemaphoreType.DMA((N_DEV,)),
        ],
        compiler_params=pltpu.CompilerParams(collective_id=0),
    )(x, w_mat, scale_x, scale_w)


# baseline (device time: 54844 ns/iter reference)
import jax
import jax.numpy as jnp
from jax import lax
from jax.experimental import pallas as pl
from jax.experimental.pallas import tpu as pltpu

N_DEV = 4
F8 = jnp.float8_e4m3fn


def kernel(x, w_mat, scale_x, scale_w):
    m_total, k_loc = x.shape
    k_total, n = w_mat.shape
    m_loc = m_total // N_DEV

    def body(x_ref, w_hbm, sx_ref, sw_ref, out_ref,
             x8_ref, recv_ref, wstage_ref, wsems, send_sems, recv_sems):
        my = lax.axis_index("i")

        dot_srcs = [my, (my + 1) % N_DEV, (my + 3) % N_DEV, (my + 2) % N_DEV]

        def wdma(i_dot):
            return pltpu.make_async_copy(
                w_hbm.at[pl.ds(dot_srcs[i_dot] * k_loc, k_loc), :],
                wstage_ref.at[i_dot % 2],
                wsems.at[i_dot % 2])

        wdma(0).start()

        for b in range(N_DEV):
            x8_ref[b, :, :] = x_ref[b * m_loc:(b + 1) * m_loc, :].astype(F8)

        barrier = pltpu.get_barrier_semaphore()
        for off in range(1, N_DEV):
            pl.semaphore_signal(
                barrier, inc=1,
                device_id=((my + off) % N_DEV,),
                device_id_type=pl.DeviceIdType.MESH)
        pl.semaphore_wait(barrier, N_DEV - 1)

        sends = []
        for off in range(1, N_DEV):
            tgt = (my + off) % N_DEV
            rdma = pltpu.make_async_remote_copy(
                src_ref=x8_ref.at[tgt],
                dst_ref=recv_ref.at[my],
                send_sem=send_sems.at[off - 1],
                recv_sem=recv_sems.at[my],
                device_id=(tgt,),
                device_id_type=pl.DeviceIdType.MESH)
            rdma.start()
            sends.append(rdma)

        wdma(1).start()

        def dot(x8_tile, i_dot):
            return lax.dot_general(
                x8_tile, wstage_ref[i_dot % 2].astype(F8),
                (((1,), (0,)), ((), ())),
                preferred_element_type=jnp.float32)

        wdma(0).wait()
        out_ref[:, :] = dot(x8_ref[my], 0)

        for i_dot in (1, 2, 3):
            src = dot_srcs[i_dot]
            recv = pltpu.make_async_remote_copy(
                src_ref=x8_ref.at[0],
                dst_ref=recv_ref.at[src],
                send_sem=send_sems.at[0],
                recv_sem=recv_sems.at[src],
                device_id=(src,),
                device_id_type=pl.DeviceIdType.MESH)
            recv.wait_recv()
            wdma(i_dot).wait()
            if i_dot < 3:
                wdma(i_dot + 1).start()
            out_ref[:, :] += dot(recv_ref[src], i_dot)

        y = out_ref[:, :] * (sx_ref[0] * sw_ref[0])
        out_ref[:, :] = y * (1.0 / (1.0 + jnp.exp(-y)))

        for rdma in sends:
            rdma.wait_send()

    return pl.pallas_call(
        body,
        out_shape=jax.ShapeDtypeStruct((m_loc, n), jnp.float32),
        in_specs=[
            pl.BlockSpec(memory_space=pltpu.VMEM),
            pl.BlockSpec(memory_space=pl.ANY),
            pl.BlockSpec(memory_space=pltpu.SMEM),
            pl.BlockSpec(memory_space=pltpu.SMEM),
        ],
        out_specs=pl.BlockSpec(memory_space=pltpu.VMEM),
        scratch_shapes=[
            pltpu.VMEM((N_DEV, m_loc, k_loc), F8),
            pltpu.VMEM((N_DEV, m_loc, k_loc), F8),
            pltpu.VMEM((2, k_loc, n), jnp.float32),
            pltpu.SemaphoreType.DMA((2,)),
            pltpu.SemaphoreType.DMA((N_DEV - 1,)),
            pltpu.SemaphoreType.DMA((N_DEV,)),
        ],
        compiler_params=pltpu.CompilerParams(
            collective_id=0, vmem_limit_bytes=60 * 1024 * 1024),
    )(x, w_mat, scale_x, scale_w)


# device time: 48548 ns/iter; 1.1297x vs baseline; 1.1297x over previous
import jax
import jax.numpy as jnp
from jax import lax
from jax.experimental import pallas as pl
from jax.experimental.pallas import tpu as pltpu

N_DEV = 4
F8 = jnp.float8_e4m3fn


def kernel(x, w_mat, scale_x, scale_w):
    m_total, k_loc = x.shape
    k_total, n = w_mat.shape
    m_loc = m_total // N_DEV

    def body(x_ref, w_hbm, sx_ref, sw_ref, out_ref,
             x8_ref, recv_ref, wstage_ref, wsems, send_sems, recv_sems):
        my = lax.axis_index("i")

        dot_srcs = [my, (my + 1) % N_DEV, (my + 3) % N_DEV, (my + 2) % N_DEV]

        def wdma(i_dot):
            return pltpu.make_async_copy(
                w_hbm.at[pl.ds(dot_srcs[i_dot] * k_loc, k_loc), :],
                wstage_ref.at[i_dot % 2],
                wsems.at[i_dot % 2])

        wdma(0).start()

        with jax.named_scope("barrier"):
            barrier = pltpu.get_barrier_semaphore()
            for off in range(1, N_DEV):
                pl.semaphore_signal(
                    barrier, inc=1,
                    device_id=((my + off) % N_DEV,),
                    device_id_type=pl.DeviceIdType.MESH)
            pl.semaphore_wait(barrier, N_DEV - 1)

        sends = []
        with jax.named_scope("cast_send"):
            for off in (1, 3, 2):
                tgt = (my + off) % N_DEV
                x8_ref[tgt, :, :] = x_ref[pl.ds(tgt * m_loc, m_loc), :].astype(F8)
                rdma = pltpu.make_async_remote_copy(
                    src_ref=x8_ref.at[tgt],
                    dst_ref=recv_ref.at[my],
                    send_sem=send_sems.at[off - 1],
                    recv_sem=recv_sems.at[my],
                    device_id=(tgt,),
                    device_id_type=pl.DeviceIdType.MESH)
                rdma.start()
                sends.append(rdma)

        wdma(1).start()

        def dot(x8_tile, i_dot):
            return lax.dot_general(
                x8_tile, wstage_ref[i_dot % 2].astype(F8),
                (((1,), (0,)), ((), ())),
                preferred_element_type=jnp.float32)

        with jax.named_scope("cast_own"):
            x8_ref[my, :, :] = x_ref[pl.ds(my * m_loc, m_loc), :].astype(F8)
        with jax.named_scope("dot0"):
            wdma(0).wait()
            out_ref[:, :] = dot(x8_ref[my], 0)

        for i_dot in (1, 2, 3):
            src = dot_srcs[i_dot]
            recv = pltpu.make_async_remote_copy(
                src_ref=x8_ref.at[0],
                dst_ref=recv_ref.at[src],
                send_sem=send_sems.at[0],
                recv_sem=recv_sems.at[src],
                device_id=(src,),
                device_id_type=pl.DeviceIdType.MESH)
            with jax.named_scope(f"wait_recv#hop={i_dot}"):
                recv.wait_recv()
                wdma(i_dot).wait()
            if i_dot < 3:
                wdma(i_dot + 1).start()
            with jax.named_scope(f"dot#hop={i_dot}"):
                out_ref[:, :] += dot(recv_ref[src], i_dot)

        with jax.named_scope("epilogue"):
            y = out_ref[:, :] * (sx_ref[0] * sw_ref[0])
            out_ref[:, :] = y * (1.0 / (1.0 + jnp.exp(-y)))

        with jax.named_scope("wait_send"):
            for rdma in sends:
                rdma.wait_send()

    return pl.pallas_call(
        body,
        out_shape=jax.ShapeDtypeStruct((m_loc, n), jnp.float32),
        in_specs=[
            pl.BlockSpec(memory_space=pltpu.VMEM),
            pl.BlockSpec(memory_space=pl.ANY),
            pl.BlockSpec(memory_space=pltpu.SMEM),
            pl.BlockSpec(memory_space=pltpu.SMEM),
        ],
        out_specs=pl.BlockSpec(memory_space=pltpu.VMEM),
        scratch_shapes=[
            pltpu.VMEM((N_DEV, m_loc, k_loc), F8),
            pltpu.VMEM((N_DEV, m_loc, k_loc), F8),
            pltpu.VMEM((2, k_loc, n), jnp.float32),
            pltpu.SemaphoreType.DMA((2,)),
            pltpu.SemaphoreType.DMA((N_DEV - 1,)),
            pltpu.SemaphoreType.DMA((N_DEV,)),
        ],
        compiler_params=pltpu.CompilerParams(
            collective_id=0, vmem_limit_bytes=60 * 1024 * 1024),
    )(x, w_mat, scale_x, scale_w)
